# baseline (speedup 1.0000x reference)
import math

import jax
import jax.numpy as jnp
from jax.experimental import pallas as pl
from jax.experimental.pallas import tpu as pltpu

_B = 32
_S = 8
_V = 65536
_H = 2048
_HH = 1024
_THRESH = 0.7
_BEAMS = 3
_INV_LOG_V = 1.0 / math.log(float(_V))
_INV_SQRT2 = 0.7071067811865476


def _main_body(lg_ref, hs_ref, w1_ref, b1_ref, w2_ref, b2_ref,
               prim_ref, conf_ref, top3_ref, mean_ref, lc_ref):
    i = pl.program_id(0)
    x = lg_ref[...]
    m = jnp.max(x, axis=1, keepdims=True)
    idx = jax.lax.broadcasted_iota(jnp.int32, (_S, _V), 1)
    t = x - m
    amax = jnp.min(jnp.where(t == 0.0, idx, _V), axis=1, keepdims=True)
    e = jnp.exp(t)
    z = jnp.sum(e, axis=1, keepdims=True)
    s1 = jnp.sum(e * t, axis=1, keepdims=True)
    entropy = jnp.log(z) - s1 / z
    max_probs = 1.0 / z
    norm_ent = entropy * _INV_LOG_V

    @pl.when(i == 0)
    def _mlp():
        h1 = jax.lax.dot_general(hs_ref[...], w1_ref[...],
                                 dimension_numbers=(((1,), (1,)), ((), ())),
                                 preferred_element_type=jnp.float32)
        h1 = h1 + b1_ref[...]
        g = 0.5 * h1 * (1.0 + jax.lax.erf(h1 * _INV_SQRT2))
        h2 = jnp.sum(g * w2_ref[...], axis=1, keepdims=True)
        lc_ref[...] = jax.nn.sigmoid(h2 + b2_ref[0])

    lc = lc_ref[pl.ds(i * _S, _S), :]
    conf = 0.4 * max_probs + 0.3 * (1.0 - norm_ent) + 0.3 * lc
    prim_ref[...] = amax.reshape(1, _S, 1)
    conf_ref[...] = conf.reshape(1, _S, 1)

    xr = x[_S - 1:_S, :].reshape(8, _V // 8)
    gidx = (jax.lax.broadcasted_iota(jnp.int32, (8, _V // 8), 0) * (_V // 8)
            + jax.lax.broadcasted_iota(jnp.int32, (8, _V // 8), 1))
    v1 = jnp.max(xr)
    i1 = jnp.min(jnp.where(xr == v1, gidx, _V))
    xr = jnp.where(gidx == i1, -jnp.inf, xr)
    v2 = jnp.max(xr)
    i2 = jnp.min(jnp.where(xr == v2, gidx, _V))
    xr = jnp.where(gidx == i2, -jnp.inf, xr)
    v3 = jnp.max(xr)
    i3 = jnp.min(jnp.where(xr == v3, gidx, _V))
    top3_ref[...] = jnp.stack([i1, i2, i3]).reshape(1, 1, _BEAMS)

    @pl.when(i == 0)
    def _init():
        mean_ref[...] = jnp.zeros((1, 1), jnp.float32)

    mean_ref[...] = mean_ref[...] + jnp.sum(conf, axis=0, keepdims=True)

    @pl.when(i == pl.num_programs(0) - 1)
    def _fin():
        mean_ref[...] = mean_ref[...] * (1.0 / (_B * _S))


def _flag_body(top3_ref, mean_ref, alt_ref):
    flag = (mean_ref[...] < _THRESH).astype(jnp.int32)
    alt_ref[...] = top3_ref[...] * flag


def kernel(model, input_ids, logits, hidden_states, W1, b1, W2, b2):
    lg = logits.reshape(_B * _S, _V)
    hs = hidden_states.reshape(_B * _S, _H).astype(jnp.bfloat16)
    w1b = W1.astype(jnp.bfloat16)
    b1r = b1.reshape(1, _HH)
    w2r = W2.reshape(1, _HH)
    b2r = b2.reshape(1)

    prim, conf, top3, mean = pl.pallas_call(
        _main_body,
        grid=(_B,),
        in_specs=[
            pl.BlockSpec((_S, _V), lambda i: (i, 0)),
            pl.BlockSpec((_B * _S, _H), lambda i: (0, 0)),
            pl.BlockSpec((_HH, _H), lambda i: (0, 0)),
            pl.BlockSpec((1, _HH), lambda i: (0, 0)),
            pl.BlockSpec((1, _HH), lambda i: (0, 0)),
            pl.BlockSpec(memory_space=pltpu.SMEM),
        ],
        out_specs=[
            pl.BlockSpec((1, _S, 1), lambda i: (i, 0, 0)),
            pl.BlockSpec((1, _S, 1), lambda i: (i, 0, 0)),
            pl.BlockSpec((1, 1, _BEAMS), lambda i: (i, 0, 0)),
            pl.BlockSpec((1, 1), lambda i: (0, 0)),
        ],
        out_shape=[
            jax.ShapeDtypeStruct((_B, _S, 1), jnp.int32),
            jax.ShapeDtypeStruct((_B, _S, 1), jnp.float32),
            jax.ShapeDtypeStruct((_B, 1, _BEAMS), jnp.int32),
            jax.ShapeDtypeStruct((1, 1), jnp.float32),
        ],
        scratch_shapes=[pltpu.VMEM((_B * _S, 1), jnp.float32)],
    )(lg, hs, w1b, b1r, w2r, b2r)

    alternatives = pl.pallas_call(
        _flag_body,
        in_specs=[
            pl.BlockSpec((_B, _BEAMS), lambda: (0, 0)),
            pl.BlockSpec((1, 1), lambda: (0, 0)),
        ],
        out_specs=pl.BlockSpec((_B, _BEAMS), lambda: (0, 0)),
        out_shape=jax.ShapeDtypeStruct((_B, _BEAMS), jnp.int32),
    )(top3.reshape(_B, _BEAMS), mean)

    return (prim.reshape(_B, _S), conf.reshape(_B, _S),
            mean.reshape(()), alternatives)


# speedup vs baseline: 1.0588x; 1.0588x over previous
import math

import jax
import jax.numpy as jnp
from jax.experimental import pallas as pl
from jax.experimental.pallas import tpu as pltpu

_B = 32
_S = 8
_V = 65536
_H = 2048
_HH = 1024
_THRESH = 0.7
_BEAMS = 3
_INV_LOG_V = 1.0 / math.log(float(_V))
_INV_SQRT2 = 0.7071067811865476


def _main_body(lg_ref, hs_ref, w1_ref, b1_ref, w2_ref, b2_ref,
               prim_ref, conf_ref, top3_ref, mean_ref, lc_ref):
    i = pl.program_id(0)
    x = lg_ref[...]
    m = jnp.max(x, axis=1, keepdims=True)
    idx = jax.lax.broadcasted_iota(jnp.int32, (_S, _V), 1)
    t = x - m
    amax = jnp.min(jnp.where(t == 0.0, idx, _V), axis=1, keepdims=True)
    e = jnp.exp(t)
    z = jnp.sum(e, axis=1, keepdims=True)
    s1 = jnp.sum(e * t, axis=1, keepdims=True)
    entropy = jnp.log(z) - s1 / z
    max_probs = 1.0 / z
    norm_ent = entropy * _INV_LOG_V

    @pl.when(i == 0)
    def _mlp():
        h1 = jax.lax.dot_general(hs_ref[...], w1_ref[...],
                                 dimension_numbers=(((1,), (1,)), ((), ())),
                                 preferred_element_type=jnp.float32)
        h1 = h1 + b1_ref[...]
        g = 0.5 * h1 * (1.0 + jax.lax.erf(h1 * _INV_SQRT2))
        h2 = jnp.sum(g * w2_ref[...], axis=1, keepdims=True)
        lc_ref[...] = jax.nn.sigmoid(h2 + b2_ref[0])

    lc = lc_ref[pl.ds(i * _S, _S), :]
    conf = 0.4 * max_probs + 0.3 * (1.0 - norm_ent) + 0.3 * lc
    prim_ref[...] = amax.reshape(1, _S, 1)
    conf_ref[...] = conf.reshape(1, _S, 1)

    xr = x[_S - 1:_S, :].reshape(8, _V // 8)
    gidx = (jax.lax.broadcasted_iota(jnp.int32, (8, _V // 8), 0) * (_V // 8)
            + jax.lax.broadcasted_iota(jnp.int32, (8, _V // 8), 1))
    v1 = jnp.max(xr)
    i1 = jnp.min(jnp.where(xr == v1, gidx, _V))
    xr = jnp.where(gidx == i1, -jnp.inf, xr)
    v2 = jnp.max(xr)
    i2 = jnp.min(jnp.where(xr == v2, gidx, _V))
    xr = jnp.where(gidx == i2, -jnp.inf, xr)
    v3 = jnp.max(xr)
    i3 = jnp.min(jnp.where(xr == v3, gidx, _V))
    top3_ref[...] = jnp.stack([i1, i2, i3]).reshape(1, 1, _BEAMS)

    @pl.when(i == 0)
    def _init():
        mean_ref[...] = jnp.zeros((1, 1), jnp.float32)

    mean_ref[...] = mean_ref[...] + jnp.sum(conf, axis=0, keepdims=True)

    @pl.when(i == pl.num_programs(0) - 1)
    def _fin():
        mean_ref[...] = mean_ref[...] * (1.0 / (_B * _S))


def _flag_body(top3_ref, mean_ref, alt_ref):
    flag = (mean_ref[...] < _THRESH).astype(jnp.int32)
    alt_ref[...] = top3_ref[...] * flag


def kernel(model, input_ids, logits, hidden_states, W1, b1, W2, b2):
    lg = logits.reshape(_B * _S, _V)
    hs = hidden_states.reshape(_B * _S, _H)
    w1b = W1
    b1r = b1.reshape(1, _HH)
    w2r = W2.reshape(1, _HH)
    b2r = b2.reshape(1)

    prim, conf, top3, mean = pl.pallas_call(
        _main_body,
        grid=(_B,),
        in_specs=[
            pl.BlockSpec((_S, _V), lambda i: (i, 0)),
            pl.BlockSpec((_B * _S, _H), lambda i: (0, 0)),
            pl.BlockSpec((_HH, _H), lambda i: (0, 0)),
            pl.BlockSpec((1, _HH), lambda i: (0, 0)),
            pl.BlockSpec((1, _HH), lambda i: (0, 0)),
            pl.BlockSpec(memory_space=pltpu.SMEM),
        ],
        out_specs=[
            pl.BlockSpec((1, _S, 1), lambda i: (i, 0, 0)),
            pl.BlockSpec((1, _S, 1), lambda i: (i, 0, 0)),
            pl.BlockSpec((1, 1, _BEAMS), lambda i: (i, 0, 0)),
            pl.BlockSpec((1, 1), lambda i: (0, 0)),
        ],
        out_shape=[
            jax.ShapeDtypeStruct((_B, _S, 1), jnp.int32),
            jax.ShapeDtypeStruct((_B, _S, 1), jnp.float32),
            jax.ShapeDtypeStruct((_B, 1, _BEAMS), jnp.int32),
            jax.ShapeDtypeStruct((1, 1), jnp.float32),
        ],
        scratch_shapes=[pltpu.VMEM((_B * _S, 1), jnp.float32)],
    )(lg, hs, w1b, b1r, w2r, b2r)

    alternatives = pl.pallas_call(
        _flag_body,
        in_specs=[
            pl.BlockSpec((_B, _BEAMS), lambda: (0, 0)),
            pl.BlockSpec((1, 1), lambda: (0, 0)),
        ],
        out_specs=pl.BlockSpec((_B, _BEAMS), lambda: (0, 0)),
        out_shape=jax.ShapeDtypeStruct((_B, _BEAMS), jnp.int32),
    )(top3.reshape(_B, _BEAMS), mean)

    return (prim.reshape(_B, _S), conf.reshape(_B, _S),
            mean.reshape(()), alternatives)


# speedup vs baseline: 1.3985x; 1.3209x over previous
import math

import jax
import jax.numpy as jnp
from jax.experimental import pallas as pl
from jax.experimental.pallas import tpu as pltpu

_B = 32
_S = 8
_V = 65536
_H = 2048
_HH = 1024
_THRESH = 0.7
_BEAMS = 3
_INV_LOG_V = 1.0 / math.log(float(_V))
_INV_SQRT2 = 0.7071067811865476


def _main_body(lg_ref, hs_ref, w1_ref, b1_ref, w2_ref, b2_ref,
               prim_ref, conf_ref, top3_ref, mean_ref):
    i = pl.program_id(0)
    x = lg_ref[...]
    m = jnp.max(x, axis=1, keepdims=True)
    idx = jax.lax.broadcasted_iota(jnp.int32, (_S, _V), 1)
    t = x - m
    amax = jnp.min(jnp.where(t == 0.0, idx, _V), axis=1, keepdims=True)
    e = jnp.exp(t)
    z = jnp.sum(e, axis=1, keepdims=True)
    s1 = jnp.sum(e * t, axis=1, keepdims=True)
    entropy = jnp.log(z) - s1 / z
    max_probs = 1.0 / z
    norm_ent = entropy * _INV_LOG_V

    h1 = jax.lax.dot_general(hs_ref[...], w1_ref[...],
                             dimension_numbers=(((1,), (1,)), ((), ())),
                             preferred_element_type=jnp.float32)
    h1 = h1 + b1_ref[...]
    g = 0.5 * h1 * (1.0 + jax.lax.erf(h1 * _INV_SQRT2))
    h2 = jnp.sum(g * w2_ref[...], axis=1, keepdims=True)
    lc = jax.nn.sigmoid(h2 + b2_ref[0])

    conf = 0.4 * max_probs + 0.3 * (1.0 - norm_ent) + 0.3 * lc
    prim_ref[...] = amax.reshape(1, _S, 1)
    conf_ref[...] = conf.reshape(1, _S, 1)

    xr = x[_S - 1:_S, :].reshape(8, _V // 8)
    gidx = (jax.lax.broadcasted_iota(jnp.int32, (8, _V // 8), 0) * (_V // 8)
            + jax.lax.broadcasted_iota(jnp.int32, (8, _V // 8), 1))
    v1 = jnp.max(xr)
    i1 = jnp.min(jnp.where(xr == v1, gidx, _V))
    xr = jnp.where(gidx == i1, -jnp.inf, xr)
    v2 = jnp.max(xr)
    i2 = jnp.min(jnp.where(xr == v2, gidx, _V))
    xr = jnp.where(gidx == i2, -jnp.inf, xr)
    v3 = jnp.max(xr)
    i3 = jnp.min(jnp.where(xr == v3, gidx, _V))
    top3_ref[...] = jnp.stack([i1, i2, i3]).reshape(1, 1, _BEAMS)

    @pl.when(i == 0)
    def _init():
        mean_ref[...] = jnp.zeros((1, 1), jnp.float32)

    mean_ref[...] = mean_ref[...] + jnp.sum(conf, axis=0, keepdims=True)

    @pl.when(i == pl.num_programs(0) - 1)
    def _fin():
        mean_ref[...] = mean_ref[...] * (1.0 / (_B * _S))


def _flag_body(top3_ref, mean_ref, alt_ref):
    flag = (mean_ref[...] < _THRESH).astype(jnp.int32)
    alt_ref[...] = top3_ref[...] * flag


def kernel(model, input_ids, logits, hidden_states, W1, b1, W2, b2):
    lg = logits.reshape(_B * _S, _V)
    hs = hidden_states.reshape(_B * _S, _H)
    w1b = W1
    b1r = b1.reshape(1, _HH)
    w2r = W2.reshape(1, _HH)
    b2r = b2.reshape(1)

    prim, conf, top3, mean = pl.pallas_call(
        _main_body,
        grid=(_B,),
        in_specs=[
            pl.BlockSpec((_S, _V), lambda i: (i, 0)),
            pl.BlockSpec((_S, _H), lambda i: (i, 0)),
            pl.BlockSpec((_HH, _H), lambda i: (0, 0)),
            pl.BlockSpec((1, _HH), lambda i: (0, 0)),
            pl.BlockSpec((1, _HH), lambda i: (0, 0)),
            pl.BlockSpec(memory_space=pltpu.SMEM),
        ],
        out_specs=[
            pl.BlockSpec((1, _S, 1), lambda i: (i, 0, 0)),
            pl.BlockSpec((1, _S, 1), lambda i: (i, 0, 0)),
            pl.BlockSpec((1, 1, _BEAMS), lambda i: (i, 0, 0)),
            pl.BlockSpec((1, 1), lambda i: (0, 0)),
        ],
        out_shape=[
            jax.ShapeDtypeStruct((_B, _S, 1), jnp.int32),
            jax.ShapeDtypeStruct((_B, _S, 1), jnp.float32),
            jax.ShapeDtypeStruct((_B, 1, _BEAMS), jnp.int32),
            jax.ShapeDtypeStruct((1, 1), jnp.float32),
        ],
    )(lg, hs, w1b, b1r, w2r, b2r)

    alternatives = pl.pallas_call(
        _flag_body,
        in_specs=[
            pl.BlockSpec((_B, _BEAMS), lambda: (0, 0)),
            pl.BlockSpec((1, 1), lambda: (0, 0)),
        ],
        out_specs=pl.BlockSpec((_B, _BEAMS), lambda: (0, 0)),
        out_shape=jax.ShapeDtypeStruct((_B, _BEAMS), jnp.int32),
    )(top3.reshape(_B, _BEAMS), mean)

    return (prim.reshape(_B, _S), conf.reshape(_B, _S),
            mean.reshape(()), alternatives)


# speedup vs baseline: 1.6510x; 1.1805x over previous
import math

import jax
import jax.numpy as jnp
from jax.experimental import pallas as pl
from jax.experimental.pallas import tpu as pltpu

_B = 32
_S = 8
_V = 65536
_H = 2048
_HH = 1024
_THRESH = 0.7
_BEAMS = 3
_R = 16
_NB = _R // _S
_INV_LOG_V = 1.0 / math.log(float(_V))
_INV_SQRT2 = 0.7071067811865476


def _main_body(lg_ref, hs_ref, w1_ref, b1_ref, w2_ref, b2_ref,
               prim_ref, conf_ref, top3_ref, mean_ref):
    i = pl.program_id(0)
    x = lg_ref[...]
    m = jnp.max(x, axis=1, keepdims=True)
    idx = jax.lax.broadcasted_iota(jnp.int32, (_R, _V), 1)
    t = x - m
    amax = jnp.min(jnp.where(t == 0.0, idx, _V), axis=1, keepdims=True)
    e = jnp.exp(t)
    z = jnp.sum(e, axis=1, keepdims=True)
    s1 = jnp.sum(e * t, axis=1, keepdims=True)
    entropy = jnp.log(z) - s1 / z
    max_probs = 1.0 / z
    norm_ent = entropy * _INV_LOG_V

    h1 = jax.lax.dot_general(hs_ref[...], w1_ref[...],
                             dimension_numbers=(((1,), (1,)), ((), ())),
                             preferred_element_type=jnp.float32)
    h1 = h1 + b1_ref[...]
    g = 0.5 * h1 * (1.0 + jax.lax.erf(h1 * _INV_SQRT2))
    h2 = jnp.sum(g * w2_ref[...], axis=1, keepdims=True)
    lc = jax.nn.sigmoid(h2 + b2_ref[0])

    conf = 0.4 * max_probs + 0.3 * (1.0 - norm_ent) + 0.3 * lc
    prim_ref[...] = amax.reshape(1, _R, 1)
    conf_ref[...] = conf.reshape(1, _R, 1)

    gidx = (jax.lax.broadcasted_iota(jnp.int32, (8, _V // 8), 0) * (_V // 8)
            + jax.lax.broadcasted_iota(jnp.int32, (8, _V // 8), 1))
    tops = []
    for k in range(_NB):
        r = 8 * k + 7
        xr = x[r:r + 1, :].reshape(8, _V // 8)
        v1 = jnp.max(xr)
        i1 = jnp.min(jnp.where(xr == v1, gidx, _V))
        xr = jnp.where(gidx == i1, -jnp.inf, xr)
        v2 = jnp.max(xr)
        i2 = jnp.min(jnp.where(xr == v2, gidx, _V))
        xr = jnp.where(gidx == i2, -jnp.inf, xr)
        v3 = jnp.max(xr)
        i3 = jnp.min(jnp.where(xr == v3, gidx, _V))
        tops += [i1, i2, i3]
    top3_ref[...] = jnp.stack(tops).reshape(1, 1, _NB * _BEAMS)

    @pl.when(i == 0)
    def _init():
        mean_ref[...] = jnp.zeros((1, 1), jnp.float32)

    mean_ref[...] = mean_ref[...] + jnp.sum(conf, axis=0, keepdims=True)

    @pl.when(i == pl.num_programs(0) - 1)
    def _fin():
        mean_ref[...] = mean_ref[...] * (1.0 / (_B * _S))


def _flag_body(top3_ref, mean_ref, alt_ref):
    flag = (mean_ref[...] < _THRESH).astype(jnp.int32)
    alt_ref[...] = top3_ref[...] * flag


def kernel(model, input_ids, logits, hidden_states, W1, b1, W2, b2):
    lg = logits.reshape(_B * _S, _V)
    hs = hidden_states.reshape(_B * _S, _H)
    b1r = b1.reshape(1, _HH)
    w2r = W2.reshape(1, _HH)
    b2r = b2.reshape(1)
    nsteps = _B * _S // _R

    prim, conf, top3, mean = pl.pallas_call(
        _main_body,
        grid=(nsteps,),
        in_specs=[
            pl.BlockSpec((_R, _V), lambda i: (i, 0)),
            pl.BlockSpec((_R, _H), lambda i: (i, 0)),
            pl.BlockSpec((_HH, _H), lambda i: (0, 0)),
            pl.BlockSpec((1, _HH), lambda i: (0, 0)),
            pl.BlockSpec((1, _HH), lambda i: (0, 0)),
            pl.BlockSpec(memory_space=pltpu.SMEM),
        ],
        out_specs=[
            pl.BlockSpec((1, _R, 1), lambda i: (i, 0, 0)),
            pl.BlockSpec((1, _R, 1), lambda i: (i, 0, 0)),
            pl.BlockSpec((1, 1, _NB * _BEAMS), lambda i: (i, 0, 0)),
            pl.BlockSpec((1, 1), lambda i: (0, 0)),
        ],
        out_shape=[
            jax.ShapeDtypeStruct((nsteps, _R, 1), jnp.int32),
            jax.ShapeDtypeStruct((nsteps, _R, 1), jnp.float32),
            jax.ShapeDtypeStruct((nsteps, 1, _NB * _BEAMS), jnp.int32),
            jax.ShapeDtypeStruct((1, 1), jnp.float32),
        ],
    )(lg, hs, W1, b1r, w2r, b2r)

    alternatives = pl.pallas_call(
        _flag_body,
        in_specs=[
            pl.BlockSpec((_B, _BEAMS), lambda: (0, 0)),
            pl.BlockSpec((1, 1), lambda: (0, 0)),
        ],
        out_specs=pl.BlockSpec((_B, _BEAMS), lambda: (0, 0)),
        out_shape=jax.ShapeDtypeStruct((_B, _BEAMS), jnp.int32),
    )(top3.reshape(_B, _BEAMS), mean)

    return (prim.reshape(_B, _S), conf.reshape(_B, _S),
            mean.reshape(()), alternatives)
